# baseline (speedup 1.0000x reference)
import jax
import jax.numpy as jnp
from jax.experimental import pallas as pl
from jax.experimental.pallas import tpu as pltpu

_NUM_STICK = 117
_NUM_TRIGGER = 99
_NUM_BUTTONS = 128
_WIDTH = 4 * _NUM_STICK + _NUM_TRIGGER + _NUM_BUTTONS
_OFFSETS = (0, _NUM_STICK, 2 * _NUM_STICK, 3 * _NUM_STICK,
            4 * _NUM_STICK, 4 * _NUM_STICK + _NUM_TRIGGER)

_TILE_SEGS = ((0, 1), (1, 2), (2, 3), (3, 4), (4, 5), (5,))

_BB = 256
_NCHUNK = 8
_CHUNK = _BB // _NCHUNK


def _chunk_copy(step, slot, c, vbuf, o_hbm, sems):
    base = step * _BB + c * _CHUNK
    return pltpu.make_async_copy(
        vbuf.at[slot, pl.ds(c * _CHUNK, _CHUNK)],
        o_hbm.at[pl.ds(base, _CHUNK)],
        sems.at[slot, c],
    )


def _onehot_body(x_ref, o_hbm, vbuf, sems):
    i = pl.program_id(0)
    nsteps = pl.num_programs(0)
    slot = jax.lax.rem(i, 2)

    @pl.when(i >= 2)
    def _():
        for c in range(_NCHUNK):
            _chunk_copy(i - 2, slot, c, vbuf, o_hbm, sems).wait()

    codes = x_ref[...]
    t = codes.shape[1]
    for k, segs in enumerate(_TILE_SEGS):
        lo = 128 * k
        hi = min(lo + 128, _WIDTH)
        w = hi - lo
        col = jax.lax.broadcasted_iota(jnp.int32, (_BB, t, w), 2) + lo
        acc = col == (codes[:, :, segs[0]:segs[0] + 1] + _OFFSETS[segs[0]])
        for s in segs[1:]:
            acc = jnp.logical_or(
                acc, col == (codes[:, :, s:s + 1] + _OFFSETS[s]))
        vbuf[slot, :, :, lo:hi] = acc.astype(jnp.float32)

    for c in range(_NCHUNK):
        _chunk_copy(i, slot, c, vbuf, o_hbm, sems).start()

    @pl.when(i == nsteps - 1)
    def _():
        for c in range(_NCHUNK):
            _chunk_copy(i - 1, 1 - slot, c, vbuf, o_hbm, sems).wait()
            _chunk_copy(i, slot, c, vbuf, o_hbm, sems).wait()


def kernel(x):
    b, t, ncomp = x.shape
    grid = (b // _BB,)
    return pl.pallas_call(
        _onehot_body,
        grid=grid,
        in_specs=[pl.BlockSpec((_BB, t, ncomp), lambda i: (i, 0, 0))],
        out_specs=pl.BlockSpec(memory_space=pl.ANY),
        out_shape=jax.ShapeDtypeStruct((b, t, _WIDTH), jnp.float32),
        scratch_shapes=[
            pltpu.VMEM((2, _BB, t, _WIDTH), jnp.float32),
            pltpu.SemaphoreType.DMA((2, _NCHUNK)),
        ],
    )(x.astype(jnp.int32))
